# baseline (speedup 1.0000x reference)
import functools

import jax
import jax.numpy as jnp
from jax import lax
from jax.experimental import pallas as pl
from jax.experimental.pallas import tpu as pltpu
from jax.experimental.pallas import tpu_sc as plsc

K = 8192
DIM = 256
B, H, W = 4, 24, 24
M = B * H * W
HW = H * W
KT = 2048
NKT = K // KT


def _argmin_body(xv_ref, lt_ref, q_ref, xf_ref, xsq_ref, minv_ref, mini_ref):
    k = pl.program_id(0)

    @pl.when(k == 0)
    def _init():
        for bb in range(B):
            xf_ref[bb * HW:(bb + 1) * HW, :] = jnp.transpose(
                xv_ref[bb], (1, 0))
        xf = xf_ref[...]
        xsq_ref[...] = jnp.sum(xf * xf, axis=1, keepdims=True)

    ltm2 = lt_ref[...] * -2.0
    csq = jnp.sum(ltm2 * ltm2, axis=1) * 0.25
    e2 = jax.lax.dot_general(
        xf_ref[...], ltm2,

        dimension_numbers=(((1,), (1,)), ((), ())),
        preferred_element_type=jnp.float32,
    )
    d2 = (xsq_ref[...] + e2) + csq[None, :]
    tmin = jnp.min(d2, axis=1, keepdims=True)
    lidx = jax.lax.broadcasted_iota(jnp.int32, (M, KT), 1).astype(jnp.float32)
    tidx = jnp.min(
        jnp.where(d2 == tmin, lidx, jnp.inf),
        axis=1, keepdims=True) + jnp.float32(k * KT)
    upd = jnp.logical_or(tmin < minv_ref[...], k == 0)
    mini_ref[...] = jnp.where(upd, tidx, mini_ref[...])
    minv_ref[...] = jnp.where(upd, tmin, minv_ref[...])

    @pl.when(k == NKT - 1)
    def _out():
        q_ref[...] = mini_ref[...].astype(jnp.int32)


def _argmin_call(xv, lt):
    return pl.pallas_call(
        _argmin_body,
        grid=(NKT,),
        in_specs=[
            pl.BlockSpec((B, DIM, HW), lambda k: (0, 0, 0)),
            pl.BlockSpec((KT, DIM), lambda k: (k, 0)),
        ],
        out_specs=[
            pl.BlockSpec((M, 1), lambda k: (0, 0)),
            pl.BlockSpec((M, DIM), lambda k: (0, 0)),
        ],
        out_shape=[
            jax.ShapeDtypeStruct((M, 1), jnp.int32),
            jax.ShapeDtypeStruct((M, DIM), jnp.float32),
        ],
        scratch_shapes=[
            pltpu.VMEM((M, 1), jnp.float32),
            pltpu.VMEM((M, 1), jnp.float32),
            pltpu.VMEM((M, 1), jnp.float32),
        ],
        compiler_params=pltpu.CompilerParams(
            dimension_semantics=("arbitrary",),
        ),
    )(xv, lt)


NW = 32
BPW = M // NW


def _sc_gather_body(q_hbm, lt_hbm, out_hbm, idx_v, rows_v, sem):
    wid = lax.axis_index("s") * 2 + lax.axis_index("c")
    base = wid * BPW
    pltpu.sync_copy(q_hbm.at[pl.ds(base, BPW)], idx_v)
    pltpu.async_copy(lt_hbm.at[idx_v], rows_v, sem).wait()
    pltpu.sync_copy(rows_v, out_hbm.at[pl.ds(base, BPW)])


_sc_gather = functools.partial(
    pl.kernel,
    mesh=plsc.VectorSubcoreMesh(core_axis_name="c", subcore_axis_name="s"),
    out_type=jax.ShapeDtypeStruct((M, DIM), jnp.float32),
    scratch_types=[
        pltpu.VMEM((BPW,), jnp.int32),
        pltpu.VMEM((BPW, DIM), jnp.float32),
        pltpu.SemaphoreType.DMA,
    ],
)(_sc_gather_body)


def _finish_body(g_ref, xr_ref, xe_ref, part_ref):
    gt = jnp.transpose(g_ref[...], (1, 0))
    xe_ref[0] = gt
    diff = xr_ref[0] - gt
    part_ref[...] = jnp.sum(diff * diff, keepdims=True)[None]


def _finish_call(g, xr4):
    return pl.pallas_call(
        _finish_body,
        grid=(B,),
        in_specs=[
            pl.BlockSpec((HW, DIM), lambda b: (b, 0)),
            pl.BlockSpec((1, DIM, HW), lambda b: (b, 0, 0)),
        ],
        out_specs=[
            pl.BlockSpec((1, DIM, HW), lambda b: (b, 0, 0)),
            pl.BlockSpec((1, 1, 1), lambda b: (b, 0, 0)),
        ],
        out_shape=[
            jax.ShapeDtypeStruct((B, DIM, HW), jnp.float32),
            jax.ShapeDtypeStruct((B, 1, 1), jnp.float32),
        ],
        compiler_params=pltpu.CompilerParams(
            dimension_semantics=("arbitrary",),
        ),
    )(g, xr4)


def kernel(x, lookup_table):
    b, d, h, w = x.shape
    lt = lookup_table[0, 0]
    xv = x.reshape(b, d, h * w)
    q2, xf = _argmin_call(xv, lt)
    q = q2[:, 0]
    x_e_flat = _sc_gather(q, lt)
    xr4 = xf.reshape(b, d, h * w)
    x_e4, parts = _finish_call(x_e_flat, xr4)
    x_e = x_e4.reshape(b, d, h, w)
    q_out = q.reshape(b, h, w)
    codebook_loss = jnp.sum(parts) / jnp.float32(b * d * h * w)
    return x_e, q_out, codebook_loss


# speedup vs baseline: 1.1383x; 1.1383x over previous
import functools

import jax
import jax.numpy as jnp
from jax import lax
from jax.experimental import pallas as pl
from jax.experimental.pallas import tpu as pltpu
from jax.experimental.pallas import tpu_sc as plsc

K = 8192
DIM = 256
B, H, W = 4, 24, 24
M = B * H * W
HW = H * W
KT = 4096
NKT = K // KT


def _argmin_body(xf_ref, lt_ref, q_ref, xsq_ref, minv_ref, mini_ref):
    k = pl.program_id(0)

    @pl.when(k == 0)
    def _init():
        xf = xf_ref[...]
        xsq_ref[...] = jnp.sum(xf * xf, axis=1, keepdims=True)

    ltm2 = lt_ref[...] * -2.0
    csq = jnp.sum(ltm2 * ltm2, axis=1) * 0.25
    e2 = jax.lax.dot_general(
        xf_ref[...], ltm2,
        dimension_numbers=(((1,), (1,)), ((), ())),
        preferred_element_type=jnp.float32,
    )
    d2 = (xsq_ref[...] + e2) + csq[None, :]
    tmin = jnp.min(d2, axis=1, keepdims=True)
    lidx = jax.lax.broadcasted_iota(jnp.int32, (M, KT), 1).astype(jnp.float32)
    tidx = jnp.min(
        jnp.where(d2 == tmin, lidx, jnp.inf),
        axis=1, keepdims=True) + jnp.float32(k * KT)
    upd = jnp.logical_or(tmin < minv_ref[...], k == 0)
    mini_ref[...] = jnp.where(upd, tidx, mini_ref[...])
    minv_ref[...] = jnp.where(upd, tmin, minv_ref[...])

    @pl.when(k == NKT - 1)
    def _out():
        q_ref[...] = mini_ref[...].astype(jnp.int32)


def _argmin_call(xf, lt):
    return pl.pallas_call(
        _argmin_body,
        grid=(NKT,),
        in_specs=[
            pl.BlockSpec((M, DIM), lambda k: (0, 0)),
            pl.BlockSpec((KT, DIM), lambda k: (k, 0)),
        ],
        out_specs=pl.BlockSpec((M, 1), lambda k: (0, 0)),
        out_shape=jax.ShapeDtypeStruct((M, 1), jnp.int32),
        scratch_shapes=[
            pltpu.VMEM((M, 1), jnp.float32),
            pltpu.VMEM((M, 1), jnp.float32),
            pltpu.VMEM((M, 1), jnp.float32),
        ],
        compiler_params=pltpu.CompilerParams(
            dimension_semantics=("arbitrary",),
        ),
    )(xf, lt)


NW = 32
BPW = M // NW


def _sc_gather_body(q_hbm, lt_hbm, out_hbm, idx_v, rows_v, sem):
    wid = lax.axis_index("s") * 2 + lax.axis_index("c")
    base = wid * BPW
    pltpu.sync_copy(q_hbm.at[pl.ds(base, BPW)], idx_v)
    pltpu.async_copy(lt_hbm.at[idx_v], rows_v, sem).wait()
    pltpu.sync_copy(rows_v, out_hbm.at[pl.ds(base, BPW)])


_sc_gather = functools.partial(
    pl.kernel,
    mesh=plsc.VectorSubcoreMesh(core_axis_name="c", subcore_axis_name="s"),
    out_type=jax.ShapeDtypeStruct((M, DIM), jnp.float32),
    scratch_types=[
        pltpu.VMEM((BPW,), jnp.int32),
        pltpu.VMEM((BPW, DIM), jnp.float32),
        pltpu.SemaphoreType.DMA,
    ],
)(_sc_gather_body)


def _finish_body(g_ref, xr_ref, xe_ref, part_ref):
    gt = jnp.transpose(g_ref[...], (1, 0))
    xe_ref[0] = gt
    diff = xr_ref[0] - gt
    part_ref[...] = jnp.sum(diff * diff, keepdims=True)[None]


def _finish_call(g, xr4):
    return pl.pallas_call(
        _finish_body,
        grid=(B,),
        in_specs=[
            pl.BlockSpec((HW, DIM), lambda b: (b, 0)),
            pl.BlockSpec((1, DIM, HW), lambda b: (b, 0, 0)),
        ],
        out_specs=[
            pl.BlockSpec((1, DIM, HW), lambda b: (b, 0, 0)),
            pl.BlockSpec((1, 1, 1), lambda b: (b, 0, 0)),
        ],
        out_shape=[
            jax.ShapeDtypeStruct((B, DIM, HW), jnp.float32),
            jax.ShapeDtypeStruct((B, 1, 1), jnp.float32),
        ],
        compiler_params=pltpu.CompilerParams(
            dimension_semantics=("arbitrary",),
        ),
    )(g, xr4)


def kernel(x, lookup_table):
    b, d, h, w = x.shape
    lt = lookup_table[0, 0]
    xf = jnp.transpose(x.reshape(b, d, h * w), (0, 2, 1)).reshape(M, DIM)
    q = _argmin_call(xf, lt)[:, 0]
    x_e_flat = _sc_gather(q, lt)
    x_e = jnp.transpose(x_e_flat.reshape(b, h * w, d), (0, 2, 1)).reshape(
        b, d, h, w)
    q_out = q.reshape(b, h, w)
    x_r = xf.reshape(b, d, h, w)
    codebook_loss = jnp.mean((x_r - x_e) ** 2)
    return x_e, q_out, codebook_loss


# speedup vs baseline: 1.2617x; 1.1084x over previous
import functools

import jax
import jax.numpy as jnp
from jax import lax
from jax.experimental import pallas as pl
from jax.experimental.pallas import tpu as pltpu
from jax.experimental.pallas import tpu_sc as plsc

K = 8192
DIM = 256
B, H, W = 4, 24, 24
M = B * H * W
HW = H * W
KT = 4096
NKT = K // KT


def _argmin_body(xf_ref, lt_ref, q_ref, xsq_ref, minv_ref, mini_ref):
    k = pl.program_id(0)

    @pl.when(k == 0)
    def _init():
        xf = xf_ref[...]
        xsq_ref[...] = jnp.sum(xf * xf, axis=1, keepdims=True)

    ltm2 = lt_ref[...] * -2.0
    csq = jnp.sum(ltm2 * ltm2, axis=1) * 0.25
    e2 = jax.lax.dot_general(
        xf_ref[...], ltm2,
        dimension_numbers=(((1,), (1,)), ((), ())),
        preferred_element_type=jnp.float32,
    )
    d2 = (xsq_ref[...] + e2) + csq[None, :]
    tmin = jnp.min(d2, axis=1, keepdims=True)
    lidx = jax.lax.broadcasted_iota(jnp.int32, (M, KT), 1).astype(jnp.float32)
    tidx = jnp.min(
        jnp.where(d2 == tmin, lidx, jnp.inf),
        axis=1, keepdims=True) + jnp.float32(k * KT)
    upd = jnp.logical_or(tmin < minv_ref[...], k == 0)
    mini_ref[...] = jnp.where(upd, tidx, mini_ref[...])
    minv_ref[...] = jnp.where(upd, tmin, minv_ref[...])

    @pl.when(k == NKT - 1)
    def _out():
        q_ref[...] = mini_ref[...].astype(jnp.int32)


def _argmin_call(xf, lt):
    return pl.pallas_call(
        _argmin_body,
        grid=(NKT,),
        in_specs=[
            pl.BlockSpec((M, DIM), lambda k: (0, 0)),
            pl.BlockSpec((KT, DIM), lambda k: (k, 0)),
        ],
        out_specs=pl.BlockSpec((M, 1), lambda k: (0, 0)),
        out_shape=jax.ShapeDtypeStruct((M, 1), jnp.int32),
        scratch_shapes=[
            pltpu.VMEM((M, 1), jnp.float32),
            pltpu.VMEM((M, 1), jnp.float32),
            pltpu.VMEM((M, 1), jnp.float32),
        ],
        compiler_params=pltpu.CompilerParams(
            dimension_semantics=("arbitrary",),
        ),
    )(xf, lt)


NW = 32
BPW = M // NW


def _sc_gather_body(q_hbm, lt_hbm, out_hbm, idx_v, rows_v, sem):
    wid = lax.axis_index("s") * 2 + lax.axis_index("c")
    base = wid * BPW
    pltpu.sync_copy(q_hbm.at[pl.ds(base, BPW)], idx_v)
    pltpu.async_copy(lt_hbm.at[idx_v], rows_v, sem).wait()
    pltpu.sync_copy(rows_v, out_hbm.at[pl.ds(base, BPW)])


_sc_gather = functools.partial(
    pl.kernel,
    mesh=plsc.VectorSubcoreMesh(core_axis_name="c", subcore_axis_name="s"),
    out_type=jax.ShapeDtypeStruct((M, DIM), jnp.float32),
    scratch_types=[
        pltpu.VMEM((BPW,), jnp.int32),
        pltpu.VMEM((BPW, DIM), jnp.float32),
        pltpu.SemaphoreType.DMA,
    ],
)(_sc_gather_body)


def _gf_body(qs_ref, lt_ref, xr_ref, xe_ref, part_ref, g_ref):
    bb = pl.program_id(0)

    def row(t, _):
        idx = qs_ref[bb * HW + t]
        g_ref[pl.ds(t, 1), :] = lt_ref[pl.ds(idx, 1), :]
        return 0

    jax.lax.fori_loop(0, HW, row, 0, unroll=8)
    gt = jnp.transpose(g_ref[...], (1, 0))
    xe_ref[0] = gt
    diff = xr_ref[0] - gt
    part_ref[...] = jnp.sum(diff * diff, keepdims=True)[None]


def _gather_finish(q, lt, xr4):
    grid_spec = pltpu.PrefetchScalarGridSpec(
        num_scalar_prefetch=1,
        grid=(B,),
        in_specs=[
            pl.BlockSpec((K, DIM), lambda b, qs: (0, 0)),
            pl.BlockSpec((1, DIM, HW), lambda b, qs: (b, 0, 0)),
        ],
        out_specs=[
            pl.BlockSpec((1, DIM, HW), lambda b, qs: (b, 0, 0)),
            pl.BlockSpec((1, 1, 1), lambda b, qs: (b, 0, 0)),
        ],
        scratch_shapes=[pltpu.VMEM((HW, DIM), jnp.float32)],
    )
    return pl.pallas_call(
        _gf_body,
        grid_spec=grid_spec,
        out_shape=[
            jax.ShapeDtypeStruct((B, DIM, HW), jnp.float32),
            jax.ShapeDtypeStruct((B, 1, 1), jnp.float32),
        ],
        compiler_params=pltpu.CompilerParams(
            dimension_semantics=("arbitrary",),
        ),
    )(q, lt, xr4)


def kernel(x, lookup_table):
    b, d, h, w = x.shape
    lt = lookup_table[0, 0]
    xf = jnp.transpose(x.reshape(b, d, h * w), (0, 2, 1)).reshape(M, DIM)
    q2 = _argmin_call(xf, lt)
    q = q2[:, 0]
    x_e4, parts = _gather_finish(q, lt, xf.reshape(b, d, h * w))
    x_e = x_e4.reshape(b, d, h, w)
    q_out = q.reshape(b, h, w)
    codebook_loss = jnp.sum(parts) / jnp.float32(b * d * h * w)
    return x_e, q_out, codebook_loss


# speedup vs baseline: 1.2939x; 1.0256x over previous
import functools

import jax
import jax.numpy as jnp
from jax import lax
from jax.experimental import pallas as pl
from jax.experimental.pallas import tpu as pltpu
from jax.experimental.pallas import tpu_sc as plsc

K = 8192
DIM = 256
B, H, W = 4, 24, 24
HW = H * W
M = B * HW
KT = 2048
NKT = K // KT


def _mega_body(xf_ref, lt_ref, xr_ref, q_ref, xe_ref, part_ref,
               xsq_ref, minv_ref, mini_ref, qi_ref, qs_ref, g_ref, dsem):
    k = pl.program_id(0)

    @pl.when(k == 0)
    def _init():
        xf = xf_ref[...]
        xsq_ref[...] = jnp.sum(xf * xf, axis=1, keepdims=True)

    @pl.when(k < NKT)
    def _argmin_step():
        ltm2 = lt_ref[pl.ds(k * KT, KT), :] * -2.0
        csq = jnp.sum(ltm2 * ltm2, axis=1) * 0.25
        e2 = jax.lax.dot_general(
            xf_ref[...], ltm2,
            dimension_numbers=(((1,), (1,)), ((), ())),
            preferred_element_type=jnp.float32,
        )
        d2 = (xsq_ref[...] + e2) + csq[None, :]
        tmin = jnp.min(d2, axis=1, keepdims=True)
        lidx = jax.lax.broadcasted_iota(
            jnp.int32, (M, KT), 1).astype(jnp.float32)
        tidx = jnp.min(
            jnp.where(d2 == tmin, lidx, jnp.inf),
            axis=1, keepdims=True) + jnp.float32(k * KT)
        upd = jnp.logical_or(tmin < minv_ref[...], k == 0)
        mini_ref[...] = jnp.where(upd, tidx, mini_ref[...])
        minv_ref[...] = jnp.where(upd, tmin, minv_ref[...])

    @pl.when(k == NKT)
    def _finish_step():
        qi = mini_ref[...].astype(jnp.int32)
        q_ref[...] = qi
        qi_ref[...] = qi[:, 0]
        copy = pltpu.make_async_copy(qi_ref, qs_ref, dsem)
        copy.start()
        copy.wait()
        acc = jnp.zeros((1, 1), jnp.float32)
        for bb in range(B):
            def row(t, carry, bb=bb):
                idx = qs_ref[bb * HW + t]
                g_ref[pl.ds(t, 1), :] = lt_ref[pl.ds(idx, 1), :]
                return carry

            lax.fori_loop(0, HW, row, 0, unroll=8)
            gt = jnp.transpose(g_ref[...], (1, 0))
            xe_ref[bb] = gt
            diff = xr_ref[bb] - gt
            acc = acc + jnp.sum(diff * diff, keepdims=True)
        part_ref[...] = acc[None]


def _mega_call(xf, lt, xr4):
    return pl.pallas_call(
        _mega_body,
        grid=(NKT + 1,),
        in_specs=[
            pl.BlockSpec((M, DIM), lambda k: (0, 0)),
            pl.BlockSpec((K, DIM), lambda k: (0, 0)),
            pl.BlockSpec((B, DIM, HW), lambda k: (0, 0, 0)),
        ],
        out_specs=[
            pl.BlockSpec((M, 1), lambda k: (0, 0)),
            pl.BlockSpec((B, DIM, HW), lambda k: (0, 0, 0)),
            pl.BlockSpec((1, 1, 1), lambda k: (0, 0, 0)),
        ],
        out_shape=[
            jax.ShapeDtypeStruct((M, 1), jnp.int32),
            jax.ShapeDtypeStruct((B, DIM, HW), jnp.float32),
            jax.ShapeDtypeStruct((1, 1, 1), jnp.float32),
        ],
        scratch_shapes=[
            pltpu.VMEM((M, 1), jnp.float32),
            pltpu.VMEM((M, 1), jnp.float32),
            pltpu.VMEM((M, 1), jnp.float32),
            pltpu.VMEM((M,), jnp.int32),
            pltpu.SMEM((M,), jnp.int32),
            pltpu.VMEM((HW, DIM), jnp.float32),
            pltpu.SemaphoreType.DMA,
        ],
        compiler_params=pltpu.CompilerParams(
            dimension_semantics=("arbitrary",),
        ),
    )(xf, lt, xr4)


NW = 32
BPW = M // NW


def _sc_gather_body(q_hbm, lt_hbm, out_hbm, idx_v, rows_v, sem):
    wid = lax.axis_index("s") * 2 + lax.axis_index("c")
    base = wid * BPW
    pltpu.sync_copy(q_hbm.at[pl.ds(base, BPW)], idx_v)
    pltpu.async_copy(lt_hbm.at[idx_v], rows_v, sem).wait()
    pltpu.sync_copy(rows_v, out_hbm.at[pl.ds(base, BPW)])


_sc_gather = functools.partial(
    pl.kernel,
    mesh=plsc.VectorSubcoreMesh(core_axis_name="c", subcore_axis_name="s"),
    out_type=jax.ShapeDtypeStruct((M, DIM), jnp.float32),
    scratch_types=[
        pltpu.VMEM((BPW,), jnp.int32),
        pltpu.VMEM((BPW, DIM), jnp.float32),
        pltpu.SemaphoreType.DMA,
    ],
)(_sc_gather_body)


def kernel(x, lookup_table):
    b, d, h, w = x.shape
    lt = lookup_table[0, 0]
    xf = jnp.transpose(x.reshape(b, d, h * w), (0, 2, 1)).reshape(M, DIM)
    q2, x_e4, part = _mega_call(xf, lt, xf.reshape(b, d, h * w))
    x_e = x_e4.reshape(b, d, h, w)
    q_out = q2[:, 0].reshape(b, h, w)
    codebook_loss = part[0, 0, 0] / jnp.float32(b * d * h * w)
    return x_e, q_out, codebook_loss


# speedup vs baseline: 1.3360x; 1.0325x over previous
import functools

import jax
import jax.numpy as jnp
from jax import lax
from jax.experimental import pallas as pl
from jax.experimental.pallas import tpu as pltpu
from jax.experimental.pallas import tpu_sc as plsc

K = 8192
DIM = 256
B, H, W = 4, 24, 24
HW = H * W
M = B * HW
KT = 4096
NKT = K // KT


def _mega_body(xf_ref, lt_ref, xr_ref, q_ref, xe_ref, part_ref,
               xsq_ref, minv_ref, mini_ref, qi_ref, qs_ref, g_ref, dsem):
    k = pl.program_id(0)

    @pl.when(k == 0)
    def _init():
        xf = xf_ref[...]
        xsq_ref[...] = jnp.sum(xf * xf, axis=1, keepdims=True)

    @pl.when(k < NKT)
    def _argmin_step():
        ltm2 = lt_ref[pl.ds(k * KT, KT), :] * -2.0
        csq = jnp.sum(ltm2 * ltm2, axis=1) * 0.25
        e2 = jax.lax.dot_general(
            xf_ref[...], ltm2,
            dimension_numbers=(((1,), (1,)), ((), ())),
            preferred_element_type=jnp.float32,
        )
        d2 = (xsq_ref[...] + e2) + csq[None, :]
        tmin = jnp.min(d2, axis=1, keepdims=True)
        lidx = jax.lax.broadcasted_iota(
            jnp.int32, (M, KT), 1).astype(jnp.float32)
        tidx = jnp.min(
            jnp.where(d2 == tmin, lidx, jnp.inf),
            axis=1, keepdims=True) + jnp.float32(k * KT)
        upd = jnp.logical_or(tmin < minv_ref[...], k == 0)
        mini_ref[...] = jnp.where(upd, tidx, mini_ref[...])
        minv_ref[...] = jnp.where(upd, tmin, minv_ref[...])

    @pl.when(k == NKT)
    def _finish_step():
        qi = mini_ref[...].astype(jnp.int32)
        q_ref[...] = qi
        qi_ref[...] = qi[:, 0]
        copy = pltpu.make_async_copy(qi_ref, qs_ref, dsem)
        copy.start()
        copy.wait()
        acc = jnp.zeros((1, 1), jnp.float32)
        for bb in range(B):
            def row(t, carry, bb=bb):
                idx = qs_ref[bb * HW + t]
                g_ref[pl.ds(t, 1), :] = lt_ref[pl.ds(idx, 1), :]
                return carry

            lax.fori_loop(0, HW, row, 0, unroll=16)
            gt = jnp.transpose(g_ref[...], (1, 0))
            xe_ref[bb] = gt
            diff = xr_ref[bb] - gt
            acc = acc + jnp.sum(diff * diff, keepdims=True)
        part_ref[...] = acc[None]


def _mega_call(xf, lt, xr4):
    return pl.pallas_call(
        _mega_body,
        grid=(NKT + 1,),
        in_specs=[
            pl.BlockSpec((M, DIM), lambda k: (0, 0)),
            pl.BlockSpec((K, DIM), lambda k: (0, 0)),
            pl.BlockSpec((B, DIM, HW), lambda k: (0, 0, 0)),
        ],
        out_specs=[
            pl.BlockSpec((M, 1), lambda k: (0, 0)),
            pl.BlockSpec((B, DIM, HW), lambda k: (0, 0, 0)),
            pl.BlockSpec((1, 1, 1), lambda k: (0, 0, 0)),
        ],
        out_shape=[
            jax.ShapeDtypeStruct((M, 1), jnp.int32),
            jax.ShapeDtypeStruct((B, DIM, HW), jnp.float32),
            jax.ShapeDtypeStruct((1, 1, 1), jnp.float32),
        ],
        scratch_shapes=[
            pltpu.VMEM((M, 1), jnp.float32),
            pltpu.VMEM((M, 1), jnp.float32),
            pltpu.VMEM((M, 1), jnp.float32),
            pltpu.VMEM((M,), jnp.int32),
            pltpu.SMEM((M,), jnp.int32),
            pltpu.VMEM((HW, DIM), jnp.float32),
            pltpu.SemaphoreType.DMA,
        ],
        compiler_params=pltpu.CompilerParams(
            dimension_semantics=("arbitrary",),
        ),
    )(xf, lt, xr4)


NW = 32
BPW = M // NW


def _sc_gather_body(q_hbm, lt_hbm, out_hbm, idx_v, rows_v, sem):
    wid = lax.axis_index("s") * 2 + lax.axis_index("c")
    base = wid * BPW
    pltpu.sync_copy(q_hbm.at[pl.ds(base, BPW)], idx_v)
    pltpu.async_copy(lt_hbm.at[idx_v], rows_v, sem).wait()
    pltpu.sync_copy(rows_v, out_hbm.at[pl.ds(base, BPW)])


_sc_gather = functools.partial(
    pl.kernel,
    mesh=plsc.VectorSubcoreMesh(core_axis_name="c", subcore_axis_name="s"),
    out_type=jax.ShapeDtypeStruct((M, DIM), jnp.float32),
    scratch_types=[
        pltpu.VMEM((BPW,), jnp.int32),
        pltpu.VMEM((BPW, DIM), jnp.float32),
        pltpu.SemaphoreType.DMA,
    ],
)(_sc_gather_body)


def kernel(x, lookup_table):
    b, d, h, w = x.shape
    lt = lookup_table[0, 0]
    xf = jnp.transpose(x.reshape(b, d, h * w), (0, 2, 1)).reshape(M, DIM)
    q2, x_e4, part = _mega_call(xf, lt, xf.reshape(b, d, h * w))
    x_e = x_e4.reshape(b, d, h, w)
    q_out = q2[:, 0].reshape(b, h, w)
    codebook_loss = part[0, 0, 0] / jnp.float32(b * d * h * w)
    return x_e, q_out, codebook_loss
